# baseline (speedup 1.0000x reference)
import functools

import jax
import jax.numpy as jnp
from jax import lax
from jax.experimental import pallas as pl
from jax.experimental.pallas import tpu as pltpu
from jax.experimental.pallas import tpu_sc as plsc

NUM_FACTORS = 32
VOCAB = 1000001
BATCH = 16384
H0, H1, H2 = 64, 32, 16

NC, NS = 2, 16
NW = NC * NS
BPW = BATCH // NW
CH = 128
NCH = BPW // CH

CBLK = 16384
QB = CBLK // 4
NBLK = pl.cdiv(VOCAB, CBLK)
NPACK = NBLK * QB
BLK = 4096


def _repack_body(*refs):
    in_refs, out_refs = refs[:4], refs[4:]
    for in_ref, out_ref in zip(in_refs, out_refs):
        x = in_ref[...]
        y = jnp.concatenate(
            [x[:, c * QB:(c + 1) * QB] for c in range(4)], axis=0)
        out_ref[...] = y.T


def _tc_repack4(t0, t1, t2, t3):
    return pl.pallas_call(
        _repack_body,
        grid=(NBLK,),
        in_specs=[pl.BlockSpec((NUM_FACTORS, CBLK), lambda i: (0, i))] * 4,
        out_specs=[pl.BlockSpec((QB, 4 * NUM_FACTORS), lambda i: (i, 0))] * 4,
        out_shape=[jax.ShapeDtypeStruct((NPACK, 4 * NUM_FACTORS),
                                        jnp.float32)] * 4,
    )(t0, t1, t2, t3)


def _sc_gather(gu, gi, Pp, Qp, Up, Vp):
    mesh = plsc.VectorSubcoreMesh(core_axis_name="c", subcore_axis_name="s")
    out_t = tuple(jax.ShapeDtypeStruct((BATCH, 4 * NUM_FACTORS), jnp.float32)
                  for _ in range(4))

    @functools.partial(
        pl.kernel, mesh=mesh, out_type=out_t,
        scratch_types=[
            pltpu.VMEM((CH,), jnp.int32),
            pltpu.VMEM((CH,), jnp.int32),
            pltpu.VMEM((CH, 4 * NUM_FACTORS), jnp.float32),
            pltpu.VMEM((CH, 4 * NUM_FACTORS), jnp.float32),
            pltpu.VMEM((CH, 4 * NUM_FACTORS), jnp.float32),
            pltpu.VMEM((CH, 4 * NUM_FACTORS), jnp.float32),
            pltpu.SemaphoreType.DMA,
        ],
    )
    def gather_kernel(gu_hbm, gi_hbm, p_hbm, q_hbm, u_hbm, v_hbm,
                      p_out, q_out, u_out, v_out,
                      uidx, iidx, pr, qr, ur, vr, sem):
        wid = lax.axis_index("s") * NC + lax.axis_index("c")
        base = wid * BPW

        def chunk(j):
            off = base + j * CH
            pltpu.sync_copy(gu_hbm.at[pl.ds(off, CH)], uidx)
            pltpu.sync_copy(gi_hbm.at[pl.ds(off, CH)], iidx)
            copies = [
                pltpu.async_copy(p_hbm.at[uidx], pr, sem),
                pltpu.async_copy(u_hbm.at[uidx], ur, sem),
                pltpu.async_copy(q_hbm.at[iidx], qr, sem),
                pltpu.async_copy(v_hbm.at[iidx], vr, sem),
            ]
            for c in copies:
                c.wait()
            sl = pl.ds(off, CH)
            pltpu.sync_copy(pr, p_out.at[sl])
            pltpu.sync_copy(qr, q_out.at[sl])
            pltpu.sync_copy(ur, u_out.at[sl])
            pltpu.sync_copy(vr, v_out.at[sl])

        for j in range(NCH):
            chunk(j)

    return gather_kernel(gu, gi, Pp, Qp, Up, Vp)


def _extract(g, off_ref):
    out = jnp.zeros((g.shape[0], NUM_FACTORS), jnp.float32)
    for c in range(4):
        sel = off_ref == (c * NUM_FACTORS)
        out = jnp.where(sel, g[:, c * NUM_FACTORS:(c + 1) * NUM_FACTORS], out)
    return out


def _mlp_body(p_ref, q_ref, u_ref, v_ref, ou_ref, oi_ref,
              w0_ref, b0_ref, w1_ref, b1_ref, w2_ref, b2_ref, wp_ref,
              out_ref):
    hi = lax.Precision.HIGHEST
    ou = ou_ref[...]
    oi = oi_ref[...]
    p = _extract(p_ref[...], ou)
    q = _extract(q_ref[...], oi)
    u = _extract(u_ref[...], ou)
    v = _extract(v_ref[...], oi)
    gmf = p * q
    w0 = w0_ref[...]
    h = (jnp.dot(u, w0[:NUM_FACTORS], precision=hi)
         + jnp.dot(v, w0[NUM_FACTORS:], precision=hi) + b0_ref[...])
    h = jnp.maximum(h, 0.0)
    h = jnp.maximum(jnp.dot(h, w1_ref[...], precision=hi) + b1_ref[...], 0.0)
    h = jnp.maximum(jnp.dot(h, w2_ref[...], precision=hi) + b2_ref[...], 0.0)
    wp = wp_ref[...]
    out_ref[...] = (jnp.dot(gmf, wp[:NUM_FACTORS], precision=hi)
                    + jnp.dot(h, wp[NUM_FACTORS:], precision=hi))


def _mlp(gp, gq, gu, gv, ou, oi, W0, b0, W1, b1, W2, b2, Wp):
    n_blk = BATCH // BLK
    row_spec = lambda d: pl.BlockSpec((BLK, d), lambda i: (i, 0))
    full = lambda s: pl.BlockSpec(s, lambda i: (0, 0))
    return pl.pallas_call(
        _mlp_body,
        grid=(n_blk,),
        in_specs=[
            row_spec(4 * NUM_FACTORS), row_spec(4 * NUM_FACTORS),
            row_spec(4 * NUM_FACTORS), row_spec(4 * NUM_FACTORS),
            row_spec(1), row_spec(1),
            full((2 * NUM_FACTORS, H0)), full((1, H0)),
            full((H0, H1)), full((1, H1)),
            full((H1, H2)), full((1, H2)),
            full((H2 + NUM_FACTORS, 1)),
        ],
        out_specs=pl.BlockSpec((BLK, 1), lambda i: (i, 0)),
        out_shape=jax.ShapeDtypeStruct((BATCH, 1), jnp.float32),
    )(gp, gq, gu, gv, ou, oi, W0, b0.reshape(1, H0), W1, b1.reshape(1, H1),
      W2, b2.reshape(1, H2), Wp)


def kernel(user_id, item_id, P, Q, U, V, W0, b0, W1, b1, W2, b2, Wp):
    uid = user_id.astype(jnp.int32)
    iid = item_id.astype(jnp.int32)
    last_full = (NBLK - 1) * CBLK
    clamp_start = VOCAB - CBLK

    def packed_coords(idx):
        tail = idx >= last_full
        b = jnp.where(tail, NBLK - 1, idx // CBLK)
        r = jnp.where(tail, idx - clamp_start, idx % CBLK)
        g = b * QB + r % QB
        off = (r // QB) * NUM_FACTORS
        return g, off

    gu, ou = packed_coords(uid)
    gi, oi = packed_coords(iid)
    Pp, Qp, Up, Vp = _tc_repack4(P.T, Q.T, U.T, V.T)
    gp, gq, gub, gvb = _sc_gather(gu, gi, Pp, Qp, Up, Vp)
    return _mlp(gp, gq, gub, gvb, ou.reshape(BATCH, 1), oi.reshape(BATCH, 1),
                W0, b0, W1, b1, W2, b2, Wp)


# speedup vs baseline: 1.0102x; 1.0102x over previous
import functools

import jax
import jax.numpy as jnp
from jax import lax
from jax.experimental import pallas as pl
from jax.experimental.pallas import tpu as pltpu
from jax.experimental.pallas import tpu_sc as plsc

NUM_FACTORS = 32
VOCAB = 1000001
BATCH = 16384
H0, H1, H2 = 64, 32, 16

NC, NS = 2, 16
NW = NC * NS
BPW = BATCH // NW
CH = 64
NCH = BPW // CH

CBLK = 16384
QB = CBLK // 4
NBLK = pl.cdiv(VOCAB, CBLK)
NPACK = NBLK * QB
BLK = 2048


def _repack_body(*refs):
    in_refs, out_refs = refs[:4], refs[4:]
    for in_ref, out_ref in zip(in_refs, out_refs):
        x = in_ref[...]
        y = jnp.concatenate(
            [x[:, c * QB:(c + 1) * QB] for c in range(4)], axis=0)
        out_ref[...] = y.T


def _tc_repack4(t0, t1, t2, t3):
    return pl.pallas_call(
        _repack_body,
        grid=(NBLK,),
        in_specs=[pl.BlockSpec((NUM_FACTORS, CBLK), lambda i: (0, i))] * 4,
        out_specs=[pl.BlockSpec((QB, 4 * NUM_FACTORS), lambda i: (i, 0))] * 4,
        out_shape=[jax.ShapeDtypeStruct((NPACK, 4 * NUM_FACTORS),
                                        jnp.float32)] * 4,
    )(t0, t1, t2, t3)


def _sc_gather(gu, gi, Pp, Qp, Up, Vp):
    mesh = plsc.VectorSubcoreMesh(core_axis_name="c", subcore_axis_name="s")
    out_t = tuple(jax.ShapeDtypeStruct((BATCH, 4 * NUM_FACTORS), jnp.float32)
                  for _ in range(4))

    buf = lambda: pltpu.VMEM((2, CH, 4 * NUM_FACTORS), jnp.float32)

    @functools.partial(
        pl.kernel, mesh=mesh, out_type=out_t,
        scratch_types=[
            pltpu.VMEM((BPW,), jnp.int32),
            pltpu.VMEM((BPW,), jnp.int32),
            buf(), buf(), buf(), buf(),
            pltpu.SemaphoreType.DMA,
            pltpu.SemaphoreType.DMA,
        ],
    )
    def gather_kernel(gu_hbm, gi_hbm, p_hbm, q_hbm, u_hbm, v_hbm,
                      p_out, q_out, u_out, v_out,
                      uidx, iidx, pr, qr, ur, vr, sg, sw):
        wid = lax.axis_index("s") * NC + lax.axis_index("c")
        base = wid * BPW
        pltpu.sync_copy(gu_hbm.at[pl.ds(base, BPW)], uidx)
        pltpu.sync_copy(gi_hbm.at[pl.ds(base, BPW)], iidx)

        def fire(j):
            b = j % 2
            iu = uidx.at[pl.ds(j * CH, CH)]
            ii = iidx.at[pl.ds(j * CH, CH)]
            return [
                pltpu.async_copy(p_hbm.at[iu], pr.at[b], sg),
                pltpu.async_copy(u_hbm.at[iu], ur.at[b], sg),
                pltpu.async_copy(q_hbm.at[ii], qr.at[b], sg),
                pltpu.async_copy(v_hbm.at[ii], vr.at[b], sg),
            ]

        def drain(j):
            b = j % 2
            sl = pl.ds(base + j * CH, CH)
            return [
                pltpu.async_copy(pr.at[b], p_out.at[sl], sw),
                pltpu.async_copy(qr.at[b], q_out.at[sl], sw),
                pltpu.async_copy(ur.at[b], u_out.at[sl], sw),
                pltpu.async_copy(vr.at[b], v_out.at[sl], sw),
            ]

        gathers = {0: fire(0)}
        writes = {}
        for j in range(NCH):
            if j + 1 < NCH:
                if j - 1 >= 0:
                    for c in writes.pop(j - 1):
                        c.wait()
                gathers[j + 1] = fire(j + 1)
            for c in gathers.pop(j):
                c.wait()
            writes[j] = drain(j)
        for cs in writes.values():
            for c in cs:
                c.wait()

    return gather_kernel(gu, gi, Pp, Qp, Up, Vp)


def _extract(g, off_ref):
    out = jnp.zeros((g.shape[0], NUM_FACTORS), jnp.float32)
    for c in range(4):
        sel = off_ref == (c * NUM_FACTORS)
        out = jnp.where(sel, g[:, c * NUM_FACTORS:(c + 1) * NUM_FACTORS], out)
    return out


def _mlp_body(p_ref, q_ref, u_ref, v_ref, ou_ref, oi_ref,
              w0_ref, b0_ref, w1_ref, b1_ref, w2_ref, b2_ref, wp_ref,
              out_ref):
    hi = lax.Precision.HIGHEST
    ou = ou_ref[...]
    oi = oi_ref[...]
    p = _extract(p_ref[...], ou)
    q = _extract(q_ref[...], oi)
    u = _extract(u_ref[...], ou)
    v = _extract(v_ref[...], oi)
    gmf = p * q
    w0 = w0_ref[...]
    h = (jnp.dot(u, w0[:NUM_FACTORS], precision=hi)
         + jnp.dot(v, w0[NUM_FACTORS:], precision=hi) + b0_ref[...])
    h = jnp.maximum(h, 0.0)
    h = jnp.maximum(jnp.dot(h, w1_ref[...], precision=hi) + b1_ref[...], 0.0)
    h = jnp.maximum(jnp.dot(h, w2_ref[...], precision=hi) + b2_ref[...], 0.0)
    wp = wp_ref[...]
    out_ref[...] = (jnp.dot(gmf, wp[:NUM_FACTORS], precision=hi)
                    + jnp.dot(h, wp[NUM_FACTORS:], precision=hi))


def _mlp(gp, gq, gu, gv, ou, oi, W0, b0, W1, b1, W2, b2, Wp):
    n_blk = BATCH // BLK
    row_spec = lambda d: pl.BlockSpec((BLK, d), lambda i: (i, 0))
    full = lambda s: pl.BlockSpec(s, lambda i: (0, 0))
    return pl.pallas_call(
        _mlp_body,
        grid=(n_blk,),
        in_specs=[
            row_spec(4 * NUM_FACTORS), row_spec(4 * NUM_FACTORS),
            row_spec(4 * NUM_FACTORS), row_spec(4 * NUM_FACTORS),
            row_spec(1), row_spec(1),
            full((2 * NUM_FACTORS, H0)), full((1, H0)),
            full((H0, H1)), full((1, H1)),
            full((H1, H2)), full((1, H2)),
            full((H2 + NUM_FACTORS, 1)),
        ],
        out_specs=pl.BlockSpec((BLK, 1), lambda i: (i, 0)),
        out_shape=jax.ShapeDtypeStruct((BATCH, 1), jnp.float32),
    )(gp, gq, gu, gv, ou, oi, W0, b0.reshape(1, H0), W1, b1.reshape(1, H1),
      W2, b2.reshape(1, H2), Wp)


def kernel(user_id, item_id, P, Q, U, V, W0, b0, W1, b1, W2, b2, Wp):
    uid = user_id.astype(jnp.int32)
    iid = item_id.astype(jnp.int32)
    last_full = (NBLK - 1) * CBLK
    clamp_start = VOCAB - CBLK

    def packed_coords(idx):
        tail = idx >= last_full
        b = jnp.where(tail, NBLK - 1, idx // CBLK)
        r = jnp.where(tail, idx - clamp_start, idx % CBLK)
        g = b * QB + r % QB
        off = (r // QB) * NUM_FACTORS
        return g, off

    gu, ou = packed_coords(uid)
    gi, oi = packed_coords(iid)
    Pp, Qp, Up, Vp = _tc_repack4(P.T, Q.T, U.T, V.T)
    gp, gq, gub, gvb = _sc_gather(gu, gi, Pp, Qp, Up, Vp)
    return _mlp(gp, gq, gub, gvb, ou.reshape(BATCH, 1), oi.reshape(BATCH, 1),
                W0, b0, W1, b1, W2, b2, Wp)
